# baseline (speedup 1.0000x reference)
import functools

import jax
import jax.numpy as jnp
from jax import lax
from jax.experimental import pallas as pl
from jax.experimental.pallas import tpu as pltpu
from jax.experimental.pallas import tpu_sc as plsc

TOKEN_NUM = 1000000
EMB_DIM = 64
B = 1024
X_MAX = 100.0
ALPHA = 0.75

NC = 2
NS = 16
L = 16
NW = NC * NS
B_PER_W = B // NW


@functools.partial(
    pl.kernel,
    mesh=plsc.VectorSubcoreMesh(core_axis_name="c", subcore_axis_name="s"),
    out_type=jax.ShapeDtypeStruct((B,), jnp.float32),
    compiler_params=pltpu.CompilerParams(needs_layout_passes=False),
    scratch_types=[
        pltpu.VMEM((B_PER_W,), jnp.int32),
        pltpu.VMEM((B_PER_W,), jnp.int32),
        pltpu.VMEM((B_PER_W, EMB_DIM), jnp.float32),
        pltpu.VMEM((B_PER_W, EMB_DIM), jnp.float32),
        pltpu.VMEM((B_PER_W,), jnp.float32),
        pltpu.VMEM((B_PER_W,), jnp.float32),
        pltpu.VMEM((B_PER_W,), jnp.float32),
        pltpu.VMEM((L * L,), jnp.float32),
        pltpu.SemaphoreType.DMA,
    ],
)
def _sc_pred(idx_i_hbm, idx_j_hbm, emb_i_hbm, emb_j_hbm, bi_hbm, bj_hbm,
             pred_hbm, idx_iv, idx_jv, rows_iv, rows_jv, bi_v, bj_v,
             pred_v, tmp_v, sem):
    wid = lax.axis_index("s") * NC + lax.axis_index("c")
    base = wid * B_PER_W
    pltpu.sync_copy(idx_i_hbm.at[pl.ds(base, B_PER_W)], idx_iv)
    pltpu.sync_copy(idx_j_hbm.at[pl.ds(base, B_PER_W)], idx_jv)
    cps = [
        pltpu.async_copy(bi_hbm.at[idx_iv], bi_v, sem),
        pltpu.async_copy(bj_hbm.at[idx_jv], bj_v, sem),
    ]
    for g in range(B_PER_W // L):
        ivec = idx_iv[pl.ds(g * L, L)]
        jvec = idx_jv[pl.ds(g * L, L)]
        for p in range(L):
            b = g * L + p
            cps.append(pltpu.async_copy(emb_i_hbm.at[ivec[p]],
                                        rows_iv.at[b], sem))
            cps.append(pltpu.async_copy(emb_j_hbm.at[jvec[p]],
                                        rows_jv.at[b], sem))
    for cp in cps:
        cp.wait()
    lanes = lax.iota(jnp.int32, L) * L
    for g in range(B_PER_W // L):
        for p in range(L):
            b = g * L + p
            prod = rows_iv[b, pl.ds(0, L)] * rows_jv[b, pl.ds(0, L)]
            for k in range(1, EMB_DIM // L):
                prod = prod + (rows_iv[b, pl.ds(k * L, L)]
                               * rows_jv[b, pl.ds(k * L, L)])
            tmp_v[pl.ds(p * L, L)] = prod
        sl = pl.ds(g * L, L)
        acc = bi_v[sl] + bj_v[sl]
        for d in range(L):
            acc = acc + plsc.load_gather(tmp_v, [lanes + d])
        pred_v[sl] = acc
    pltpu.sync_copy(pred_v, pred_hbm.at[pl.ds(base, B_PER_W)])


_ROW_BLKS = 8
_ROWS = B // _ROW_BLKS


def _tc_outer_body(pred_ref, xij_ref, out_ref):
    xf = xij_ref[:, :].astype(jnp.float32)
    logx = jnp.log(xf)
    fx = jnp.where(xf >= X_MAX, jnp.float32(1.0),
                   jnp.exp(ALPHA * jnp.log(xf / X_MAX)))
    diff = pred_ref[:, :] - logx
    out_ref[:, :] = fx * diff * diff


_tc_outer = pl.pallas_call(
    _tc_outer_body,
    grid=(_ROW_BLKS,),
    in_specs=[
        pl.BlockSpec((_ROWS, 1), lambda i: (i, 0)),
        pl.BlockSpec((1, B), lambda i: (0, 0)),
    ],
    out_specs=pl.BlockSpec((_ROWS, B), lambda i: (i, 0)),
    out_shape=jax.ShapeDtypeStruct((B, B), jnp.float32),
)


def kernel(x, emb_i, emb_j, bi, bj):
    idx_i = x[:, 0]
    idx_j = x[:, 1]
    xij = x[:, 2]
    pred = _sc_pred(idx_i, idx_j, emb_i, emb_j,
                    bi.reshape(TOKEN_NUM), bj.reshape(TOKEN_NUM))
    out = _tc_outer(pred.reshape(B, 1), xij.reshape(1, B))
    return out.reshape(B, 1, B)


# speedup vs baseline: 12.2071x; 12.2071x over previous
import functools

import jax
import jax.numpy as jnp
from jax import lax
from jax.experimental import pallas as pl
from jax.experimental.pallas import tpu as pltpu
from jax.experimental.pallas import tpu_sc as plsc

TOKEN_NUM = 1000000
EMB_DIM = 64
B = 1024
X_MAX = 100.0
ALPHA = 0.75

NC = 2
NS = 16
L = 16
NW = NC * NS
B_PER_W = B // NW
LANE_T = 128
CHUNK = 4


@functools.partial(
    pl.kernel,
    mesh=plsc.VectorSubcoreMesh(core_axis_name="c", subcore_axis_name="s"),
    out_type=jax.ShapeDtypeStruct((B,), jnp.float32),
    compiler_params=pltpu.CompilerParams(needs_layout_passes=False),
    scratch_types=[
        pltpu.VMEM((B_PER_W,), jnp.int32),
        pltpu.VMEM((B_PER_W,), jnp.int32),
        pltpu.VMEM((CHUNK, EMB_DIM, LANE_T), jnp.float32),
        pltpu.VMEM((CHUNK, EMB_DIM, LANE_T), jnp.float32),
        pltpu.VMEM((B_PER_W, LANE_T), jnp.float32),
        pltpu.VMEM((B_PER_W, LANE_T), jnp.float32),
        pltpu.VMEM((L * L,), jnp.float32),
        pltpu.VMEM((B_PER_W,), jnp.float32),
        pltpu.SemaphoreType.DMA,
    ],
)
def _sc_pred(xt_hbm, emb_it_hbm, emb_jt_hbm, bit_hbm, bjt_hbm,
             pred_hbm, idx_iv, idx_jv, slab_i, slab_j, bslab_i, bslab_j,
             tmp_v, pred_v, sem):
    wid = lax.axis_index("s") * NC + lax.axis_index("c")
    base = wid * B_PER_W
    pltpu.sync_copy(xt_hbm.at[0, pl.ds(base, B_PER_W)], idx_iv)
    pltpu.sync_copy(xt_hbm.at[1, pl.ds(base, B_PER_W)], idx_jv)

    ivecs = [idx_iv[pl.ds(g * L, L)] for g in range(B_PER_W // L)]
    jvecs = [idx_jv[pl.ds(g * L, L)] for g in range(B_PER_W // L)]

    bias_cps = []
    for g in range(B_PER_W // L):
        iv, jv = ivecs[g], jvecs[g]
        for p in range(L):
            b = g * L + p
            ib = pl.multiple_of((iv[p] >> 7) << 7, LANE_T)
            jb = pl.multiple_of((jv[p] >> 7) << 7, LANE_T)
            bias_cps.append(pltpu.async_copy(
                bit_hbm.at[0, pl.ds(ib, LANE_T)], bslab_i.at[b], sem))
            bias_cps.append(pltpu.async_copy(
                bjt_hbm.at[0, pl.ds(jb, LANE_T)], bslab_j.at[b], sem))
    for cp in bias_cps:
        cp.wait()

    for g in range(B_PER_W // L):
        iv, jv = ivecs[g], jvecs[g]
        for c0 in range(0, L, CHUNK):
            cps = []
            for k in range(CHUNK):
                p = c0 + k
                ib = pl.multiple_of((iv[p] >> 7) << 7, LANE_T)
                jb = pl.multiple_of((jv[p] >> 7) << 7, LANE_T)
                cps.append(pltpu.async_copy(
                    emb_it_hbm.at[:, pl.ds(ib, LANE_T)], slab_i.at[k], sem))
                cps.append(pltpu.async_copy(
                    emb_jt_hbm.at[:, pl.ds(jb, LANE_T)], slab_j.at[k], sem))
            for cp in cps:
                cp.wait()
            for k in range(CHUNK):
                p = c0 + k
                slot = jnp.full((L,), k, jnp.int32)
                li = jnp.broadcast_to(iv[p] & (LANE_T - 1), (L,))
                lj = jnp.broadcast_to(jv[p] & (LANE_T - 1), (L,))
                prod = None
                for d0 in range(0, EMB_DIM, L):
                    dvec = lax.iota(jnp.int32, L) + d0
                    a = plsc.load_gather(slab_i, [slot, dvec, li])
                    bb = plsc.load_gather(slab_j, [slot, dvec, lj])
                    prod = a * bb if prod is None else prod + a * bb
                tmp_v[pl.ds(p * L, L)] = prod

        rows = lax.iota(jnp.int32, L) * L
        acc = plsc.load_gather(bslab_i, [lax.iota(jnp.int32, L) + g * L,
                                         iv & (LANE_T - 1)])
        acc = acc + plsc.load_gather(bslab_j, [lax.iota(jnp.int32, L) + g * L,
                                               jv & (LANE_T - 1)])
        for d in range(L):
            acc = acc + plsc.load_gather(tmp_v, [rows + d])
        pred_v[pl.ds(g * L, L)] = acc

    pltpu.sync_copy(pred_v, pred_hbm.at[pl.ds(base, B_PER_W)])


_ROW_BLKS = 8
_ROWS = B // _ROW_BLKS


def _tc_outer_body(pred_ref, xij_ref, out_ref):
    xf = xij_ref[:, :].astype(jnp.float32)
    logx = jnp.log(xf)
    fx = jnp.where(xf >= X_MAX, jnp.float32(1.0),
                   jnp.exp(ALPHA * jnp.log(xf / X_MAX)))
    diff = pred_ref[:, :] - logx
    out_ref[:, 0, :] = fx * diff * diff


_tc_outer = pl.pallas_call(
    _tc_outer_body,
    grid=(_ROW_BLKS,),
    in_specs=[
        pl.BlockSpec((_ROWS, 1), lambda i: (i, 0)),
        pl.BlockSpec((1, B), lambda i: (0, 0)),
    ],
    out_specs=pl.BlockSpec((_ROWS, 1, B), lambda i: (i, 0, 0)),
    out_shape=jax.ShapeDtypeStruct((B, 1, B), jnp.float32),
)


def kernel(x, emb_i, emb_j, bi, bj):
    xij = x[:, 2]
    pred = _sc_pred(x.T, emb_i.T, emb_j.T, bi.T, bj.T)
    return _tc_outer(pred.reshape(B, 1), xij.reshape(1, B))


# speedup vs baseline: 16.7210x; 1.3698x over previous
import functools

import jax
import jax.numpy as jnp
from jax import lax
from jax.experimental import pallas as pl
from jax.experimental.pallas import tpu as pltpu
from jax.experimental.pallas import tpu_sc as plsc

TOKEN_NUM = 1000000
EMB_DIM = 64
B = 1024
X_MAX = 100.0
ALPHA = 0.75

NC = 2
NS = 16
L = 16
NW = NC * NS
B_PER_W = B // NW
LANE_T = 128
CHUNK = 4


@functools.partial(
    pl.kernel,
    mesh=plsc.VectorSubcoreMesh(core_axis_name="c", subcore_axis_name="s"),
    out_type=jax.ShapeDtypeStruct((B,), jnp.float32),
    compiler_params=pltpu.CompilerParams(needs_layout_passes=False),
    scratch_types=[
        pltpu.VMEM((B_PER_W,), jnp.int32),
        pltpu.VMEM((B_PER_W,), jnp.int32),
        pltpu.VMEM((CHUNK, EMB_DIM, LANE_T), jnp.float32),
        pltpu.VMEM((CHUNK, EMB_DIM, LANE_T), jnp.float32),
        pltpu.VMEM((B_PER_W, LANE_T), jnp.float32),
        pltpu.VMEM((B_PER_W, LANE_T), jnp.float32),
        pltpu.VMEM((L * L,), jnp.float32),
        pltpu.VMEM((B_PER_W,), jnp.float32),
        pltpu.SemaphoreType.DMA,
    ],
)
def _sc_pred(xt_hbm, emb_it_hbm, emb_jt_hbm, bit_hbm, bjt_hbm,
             pred_hbm, idx_iv, idx_jv, slab_i, slab_j, bslab_i, bslab_j,
             tmp_v, pred_v, sem):
    wid = lax.axis_index("s") * NC + lax.axis_index("c")
    base = wid * B_PER_W
    pltpu.sync_copy(xt_hbm.at[0, pl.ds(base, B_PER_W)], idx_iv)
    pltpu.sync_copy(xt_hbm.at[1, pl.ds(base, B_PER_W)], idx_jv)

    ivecs = [idx_iv[pl.ds(g * L, L)] for g in range(B_PER_W // L)]
    jvecs = [idx_jv[pl.ds(g * L, L)] for g in range(B_PER_W // L)]

    bias_cps = []
    for g in range(B_PER_W // L):
        iv, jv = ivecs[g], jvecs[g]
        for p in range(L):
            b = g * L + p
            ib = pl.multiple_of((iv[p] >> 7) << 7, LANE_T)
            jb = pl.multiple_of((jv[p] >> 7) << 7, LANE_T)
            bias_cps.append(pltpu.async_copy(
                bit_hbm.at[0, pl.ds(ib, LANE_T)], bslab_i.at[b], sem))
            bias_cps.append(pltpu.async_copy(
                bjt_hbm.at[0, pl.ds(jb, LANE_T)], bslab_j.at[b], sem))
    for cp in bias_cps:
        cp.wait()

    for g in range(B_PER_W // L):
        iv, jv = ivecs[g], jvecs[g]
        for c0 in range(0, L, CHUNK):
            cps = []
            slots_i, slots_j = [], []
            ti_prev = tj_prev = None
            for k in range(CHUNK):
                p = c0 + k
                ti = iv[p] >> 7
                tj = jv[p] >> 7
                ib = pl.multiple_of(ti << 7, LANE_T)
                jb = pl.multiple_of(tj << 7, LANE_T)
                cp_i = pltpu.make_async_copy(
                    emb_it_hbm.at[:, pl.ds(ib, LANE_T)], slab_i.at[k], sem)
                cp_j = pltpu.make_async_copy(
                    emb_jt_hbm.at[:, pl.ds(jb, LANE_T)], slab_j.at[k], sem)
                if k == 0:
                    cp_i.start()
                    cp_j.start()
                    cps.append((None, cp_i))
                    cps.append((None, cp_j))
                    slot_i = jnp.int32(0)
                    slot_j = jnp.int32(0)
                else:
                    new_i = ti != ti_prev
                    new_j = tj != tj_prev

                    @pl.when(new_i)
                    def _(cp=cp_i):
                        cp.start()

                    @pl.when(new_j)
                    def _(cp=cp_j):
                        cp.start()

                    cps.append((new_i, cp_i))
                    cps.append((new_j, cp_j))
                    slot_i = jnp.where(new_i, jnp.int32(k), slots_i[-1])
                    slot_j = jnp.where(new_j, jnp.int32(k), slots_j[-1])
                slots_i.append(slot_i)
                slots_j.append(slot_j)
                ti_prev, tj_prev = ti, tj
            for cond, cp in cps:
                if cond is None:
                    cp.wait()
                else:
                    @pl.when(cond)
                    def _(cp=cp):
                        cp.wait()
            for k in range(CHUNK):
                p = c0 + k
                sl_i = jnp.broadcast_to(slots_i[k], (L,))
                sl_j = jnp.broadcast_to(slots_j[k], (L,))
                li = jnp.broadcast_to(iv[p] & (LANE_T - 1), (L,))
                lj = jnp.broadcast_to(jv[p] & (LANE_T - 1), (L,))
                prod = None
                for d0 in range(0, EMB_DIM, L):
                    dvec = lax.iota(jnp.int32, L) + d0
                    a = plsc.load_gather(slab_i, [sl_i, dvec, li])
                    bb = plsc.load_gather(slab_j, [sl_j, dvec, lj])
                    prod = a * bb if prod is None else prod + a * bb
                tmp_v[pl.ds(p * L, L)] = prod

        rows = lax.iota(jnp.int32, L) * L
        acc = plsc.load_gather(bslab_i, [lax.iota(jnp.int32, L) + g * L,
                                         iv & (LANE_T - 1)])
        acc = acc + plsc.load_gather(bslab_j, [lax.iota(jnp.int32, L) + g * L,
                                               jv & (LANE_T - 1)])
        for d in range(L):
            acc = acc + plsc.load_gather(tmp_v, [rows + d])
        pred_v[pl.ds(g * L, L)] = acc

    pltpu.sync_copy(pred_v, pred_hbm.at[pl.ds(base, B_PER_W)])


_ROW_BLKS = 8
_ROWS = B // _ROW_BLKS


def _tc_outer_body(pred_ref, xij_ref, out_ref):
    xf = xij_ref[:, :].astype(jnp.float32)
    logx = jnp.log(xf)
    fx = jnp.where(xf >= X_MAX, jnp.float32(1.0),
                   jnp.exp(ALPHA * jnp.log(xf / X_MAX)))
    diff = pred_ref[:, :] - logx
    out_ref[:, 0, :] = fx * diff * diff


_tc_outer = pl.pallas_call(
    _tc_outer_body,
    grid=(_ROW_BLKS,),
    in_specs=[
        pl.BlockSpec((_ROWS, 1), lambda i: (i, 0)),
        pl.BlockSpec((1, B), lambda i: (0, 0)),
    ],
    out_specs=pl.BlockSpec((_ROWS, 1, B), lambda i: (i, 0, 0)),
    out_shape=jax.ShapeDtypeStruct((B, 1, B), jnp.float32),
)


def kernel(x, emb_i, emb_j, bi, bj):
    xij = x[:, 2]
    pred = _sc_pred(x.T, emb_i.T, emb_j.T, bi.T, bj.T)
    return _tc_outer(pred.reshape(B, 1), xij.reshape(1, B))


# speedup vs baseline: 20.0228x; 1.1975x over previous
import functools

import jax
import jax.numpy as jnp
from jax import lax
from jax.experimental import pallas as pl
from jax.experimental.pallas import tpu as pltpu
from jax.experimental.pallas import tpu_sc as plsc

TOKEN_NUM = 1000000
EMB_DIM = 64
B = 1024
X_MAX = 100.0
ALPHA = 0.75

NC = 2
NS = 16
L = 16
NW = NC * NS
B_PER_W = B // NW
LANE_T = 128
NSLOT_R = 3
FB = NSLOT_R


@functools.partial(
    pl.kernel,
    mesh=plsc.VectorSubcoreMesh(core_axis_name="c", subcore_axis_name="s"),
    out_type=jax.ShapeDtypeStruct((B,), jnp.float32),
    compiler_params=pltpu.CompilerParams(needs_layout_passes=False),
    scratch_types=[
        pltpu.VMEM((B_PER_W,), jnp.int32),
        pltpu.VMEM((B_PER_W,), jnp.int32),
        pltpu.VMEM((NSLOT_R + 1, EMB_DIM, LANE_T), jnp.float32),
        pltpu.VMEM((NSLOT_R + 1, EMB_DIM, LANE_T), jnp.float32),
        pltpu.VMEM((B_PER_W, LANE_T), jnp.float32),
        pltpu.VMEM((B_PER_W, LANE_T), jnp.float32),
        pltpu.VMEM((L * L,), jnp.float32),
        pltpu.VMEM((B_PER_W,), jnp.float32),
        pltpu.SemaphoreType.DMA,
    ],
)
def _sc_pred(xt_hbm, emb_it_hbm, emb_jt_hbm, bit_hbm, bjt_hbm,
             pred_hbm, idx_iv, idx_jv, slab_i, slab_j, bslab_i, bslab_j,
             tmp_v, pred_v, sem):
    wid = lax.axis_index("s") * NC + lax.axis_index("c")
    base = wid * B_PER_W
    pltpu.sync_copy(xt_hbm.at[0, pl.ds(base, B_PER_W)], idx_iv)
    pltpu.sync_copy(xt_hbm.at[1, pl.ds(base, B_PER_W)], idx_jv)

    ivecs = [idx_iv[pl.ds(g * L, L)] for g in range(B_PER_W // L)]
    jvecs = [idx_jv[pl.ds(g * L, L)] for g in range(B_PER_W // L)]

    bias_cps = []
    for g in range(B_PER_W // L):
        iv, jv = ivecs[g], jvecs[g]
        for p in range(L):
            b = g * L + p
            ib = pl.multiple_of((iv[p] >> 7) << 7, LANE_T)
            jb = pl.multiple_of((jv[p] >> 7) << 7, LANE_T)
            bias_cps.append(pltpu.async_copy(
                bit_hbm.at[0, pl.ds(ib, LANE_T)], bslab_i.at[b], sem))
            bias_cps.append(pltpu.async_copy(
                bjt_hbm.at[0, pl.ds(jb, LANE_T)], bslab_j.at[b], sem))
    for cp in bias_cps:
        cp.wait()

    tis = [v >> 7 for v in ivecs]
    tjs = [v >> 7 for v in jvecs]
    last_base = jnp.int32((TOKEN_NUM - 1) // LANE_T - (NSLOT_R - 1))
    base_i = jnp.minimum(jnp.minimum(jnp.min(tis[0]), jnp.min(tis[1])),
                         last_base)
    base_j = jnp.minimum(jnp.minimum(jnp.min(tjs[0]), jnp.min(tjs[1])),
                         last_base)
    range_cps = []
    for s in range(NSLOT_R):
        ib = pl.multiple_of((base_i + s) << 7, LANE_T)
        jb = pl.multiple_of((base_j + s) << 7, LANE_T)
        range_cps.append(pltpu.async_copy(
            emb_it_hbm.at[:, pl.ds(ib, LANE_T)], slab_i.at[s], sem))
        range_cps.append(pltpu.async_copy(
            emb_jt_hbm.at[:, pl.ds(jb, LANE_T)], slab_j.at[s], sem))
    for cp in range_cps:
        cp.wait()

    for g in range(B_PER_W // L):
        iv, jv = ivecs[g], jvecs[g]
        for p in range(L):
            off_i = (iv[p] >> 7) - base_i
            off_j = (jv[p] >> 7) - base_j
            inr_i = off_i < NSLOT_R
            inr_j = off_j < NSLOT_R

            @pl.when(jnp.logical_not(inr_i))
            def _(idx=iv[p]):
                ib = pl.multiple_of((idx >> 7) << 7, LANE_T)
                pltpu.sync_copy(emb_it_hbm.at[:, pl.ds(ib, LANE_T)],
                                slab_i.at[FB])

            @pl.when(jnp.logical_not(inr_j))
            def _(idx=jv[p]):
                jb = pl.multiple_of((idx >> 7) << 7, LANE_T)
                pltpu.sync_copy(emb_jt_hbm.at[:, pl.ds(jb, LANE_T)],
                                slab_j.at[FB])

            sl_i = jnp.broadcast_to(
                jnp.where(inr_i, off_i, jnp.int32(FB)), (L,))
            sl_j = jnp.broadcast_to(
                jnp.where(inr_j, off_j, jnp.int32(FB)), (L,))
            li = jnp.broadcast_to(iv[p] & (LANE_T - 1), (L,))
            lj = jnp.broadcast_to(jv[p] & (LANE_T - 1), (L,))
            prod = None
            for d0 in range(0, EMB_DIM, L):
                dvec = lax.iota(jnp.int32, L) + d0
                a = plsc.load_gather(slab_i, [sl_i, dvec, li])
                bb = plsc.load_gather(slab_j, [sl_j, dvec, lj])
                prod = a * bb if prod is None else prod + a * bb
            tmp_v[pl.ds(p * L, L)] = prod

        rows = lax.iota(jnp.int32, L) * L
        acc = plsc.load_gather(bslab_i, [lax.iota(jnp.int32, L) + g * L,
                                         iv & (LANE_T - 1)])
        acc = acc + plsc.load_gather(bslab_j, [lax.iota(jnp.int32, L) + g * L,
                                               jv & (LANE_T - 1)])
        for d in range(L):
            acc = acc + plsc.load_gather(tmp_v, [rows + d])
        pred_v[pl.ds(g * L, L)] = acc

    pltpu.sync_copy(pred_v, pred_hbm.at[pl.ds(base, B_PER_W)])


_ROW_BLKS = 8
_ROWS = B // _ROW_BLKS


def _tc_outer_body(pred_ref, xij_ref, out_ref):
    xf = xij_ref[:, :].astype(jnp.float32)
    logx = jnp.log(xf)
    fx = jnp.where(xf >= X_MAX, jnp.float32(1.0),
                   jnp.exp(ALPHA * jnp.log(xf / X_MAX)))
    diff = pred_ref[:, :] - logx
    out_ref[:, 0, :] = fx * diff * diff


_tc_outer = pl.pallas_call(
    _tc_outer_body,
    grid=(_ROW_BLKS,),
    in_specs=[
        pl.BlockSpec((_ROWS, 1), lambda i: (i, 0)),
        pl.BlockSpec((1, B), lambda i: (0, 0)),
    ],
    out_specs=pl.BlockSpec((_ROWS, 1, B), lambda i: (i, 0, 0)),
    out_shape=jax.ShapeDtypeStruct((B, 1, B), jnp.float32),
)


def kernel(x, emb_i, emb_j, bi, bj):
    xij = x[:, 2]
    pred = _sc_pred(x.T, emb_i.T, emb_j.T, bi.T, bj.T)
    return _tc_outer(pred.reshape(B, 1), xij.reshape(1, B))


# speedup vs baseline: 21.7834x; 1.0879x over previous
import functools

import jax
import jax.numpy as jnp
from jax import lax
from jax.experimental import pallas as pl
from jax.experimental.pallas import tpu as pltpu
from jax.experimental.pallas import tpu_sc as plsc

TOKEN_NUM = 1000000
EMB_DIM = 64
B = 1024
X_MAX = 100.0
ALPHA = 0.75

NC = 2
NS = 16
L = 16
NW = NC * NS
B_PER_W = B // NW
LANE_T = 128
NSLOT_R = 3
FB = NSLOT_R


@functools.partial(
    pl.kernel,
    mesh=plsc.VectorSubcoreMesh(core_axis_name="c", subcore_axis_name="s"),
    out_type=jax.ShapeDtypeStruct((B,), jnp.float32),
    compiler_params=pltpu.CompilerParams(needs_layout_passes=False),
    scratch_types=[
        pltpu.VMEM((B_PER_W,), jnp.int32),
        pltpu.VMEM((B_PER_W,), jnp.int32),
        pltpu.VMEM((NSLOT_R + 1, EMB_DIM, LANE_T), jnp.float32),
        pltpu.VMEM((NSLOT_R + 1, EMB_DIM, LANE_T), jnp.float32),
        pltpu.VMEM((NSLOT_R + 1, LANE_T), jnp.float32),
        pltpu.VMEM((NSLOT_R + 1, LANE_T), jnp.float32),
        pltpu.VMEM((L * L,), jnp.float32),
        pltpu.VMEM((B_PER_W,), jnp.float32),
        pltpu.SemaphoreType.DMA,
    ],
)
def _sc_pred(xt_hbm, emb_it_hbm, emb_jt_hbm, bit_hbm, bjt_hbm,
             pred_hbm, idx_iv, idx_jv, slab_i, slab_j, bslab_i, bslab_j,
             tmp_v, pred_v, sem):
    wid = lax.axis_index("s") * NC + lax.axis_index("c")
    base = wid * B_PER_W
    icp = pltpu.make_async_copy(xt_hbm.at[0, pl.ds(base, B_PER_W)],
                                idx_iv, sem)
    jcp = pltpu.make_async_copy(xt_hbm.at[1, pl.ds(base, B_PER_W)],
                                idx_jv, sem)
    icp.start()
    jcp.start()
    icp.wait()
    jcp.wait()

    ivecs = [idx_iv[pl.ds(g * L, L)] for g in range(B_PER_W // L)]
    jvecs = [idx_jv[pl.ds(g * L, L)] for g in range(B_PER_W // L)]

    tis = [v >> 7 for v in ivecs]
    tjs = [v >> 7 for v in jvecs]
    last_base = jnp.int32((TOKEN_NUM - 1) // LANE_T - (NSLOT_R - 1))
    base_i = jnp.minimum(jnp.minimum(jnp.min(tis[0]), jnp.min(tis[1])),
                         last_base)
    base_j = jnp.minimum(jnp.minimum(jnp.min(tjs[0]), jnp.min(tjs[1])),
                         last_base)
    range_cps = []
    for s in range(NSLOT_R):
        ib = pl.multiple_of((base_i + s) << 7, LANE_T)
        jb = pl.multiple_of((base_j + s) << 7, LANE_T)
        range_cps.append(pltpu.async_copy(
            emb_it_hbm.at[:, pl.ds(ib, LANE_T)], slab_i.at[s], sem))
        range_cps.append(pltpu.async_copy(
            emb_jt_hbm.at[:, pl.ds(jb, LANE_T)], slab_j.at[s], sem))
        range_cps.append(pltpu.async_copy(
            bit_hbm.at[0, pl.ds(ib, LANE_T)], bslab_i.at[s], sem))
        range_cps.append(pltpu.async_copy(
            bjt_hbm.at[0, pl.ds(jb, LANE_T)], bslab_j.at[s], sem))
    for cp in range_cps:
        cp.wait()

    lane_idx = lax.iota(jnp.int32, L)
    for g in range(B_PER_W // L):
        iv, jv = ivecs[g], jvecs[g]
        off_vi = tis[g] - base_i
        off_vj = tjs[g] - base_j
        inr_vi = off_vi < NSLOT_R
        inr_vj = off_vj < NSLOT_R
        bias_fix_i = jnp.zeros((L,), jnp.float32)
        bias_fix_j = jnp.zeros((L,), jnp.float32)
        for p in range(L):
            off_i = off_vi[p]
            off_j = off_vj[p]
            inr_i = off_i < NSLOT_R
            inr_j = off_j < NSLOT_R
            li = jnp.broadcast_to(iv[p] & (LANE_T - 1), (L,))
            lj = jnp.broadcast_to(jv[p] & (LANE_T - 1), (L,))

            @pl.when(jnp.logical_not(inr_i))
            def _(idx=iv[p]):
                ib = pl.multiple_of((idx >> 7) << 7, LANE_T)
                pltpu.sync_copy(emb_it_hbm.at[:, pl.ds(ib, LANE_T)],
                                slab_i.at[FB])
                pltpu.sync_copy(bit_hbm.at[0, pl.ds(ib, LANE_T)],
                                bslab_i.at[FB])

            @pl.when(jnp.logical_not(inr_j))
            def _(idx=jv[p]):
                jb = pl.multiple_of((idx >> 7) << 7, LANE_T)
                pltpu.sync_copy(emb_jt_hbm.at[:, pl.ds(jb, LANE_T)],
                                slab_j.at[FB])
                pltpu.sync_copy(bjt_hbm.at[0, pl.ds(jb, LANE_T)],
                                bslab_j.at[FB])

            take_i = jnp.logical_and(lane_idx == p,
                                     jnp.broadcast_to(~inr_i, (L,)))
            take_j = jnp.logical_and(lane_idx == p,
                                     jnp.broadcast_to(~inr_j, (L,)))
            fbs = jnp.full((L,), FB, jnp.int32)
            bias_fix_i = jnp.where(
                take_i, plsc.load_gather(bslab_i, [fbs, li]), bias_fix_i)
            bias_fix_j = jnp.where(
                take_j, plsc.load_gather(bslab_j, [fbs, lj]), bias_fix_j)

            sl_i = jnp.broadcast_to(
                jnp.where(inr_i, off_i, jnp.int32(FB)), (L,))
            sl_j = jnp.broadcast_to(
                jnp.where(inr_j, off_j, jnp.int32(FB)), (L,))
            prod = None
            for d0 in range(0, EMB_DIM, L):
                dvec = lane_idx + d0
                a = plsc.load_gather(slab_i, [sl_i, dvec, li])
                bb = plsc.load_gather(slab_j, [sl_j, dvec, lj])
                prod = a * bb if prod is None else prod + a * bb
            tmp_v[pl.ds(p * L, L)] = prod

        slot_vi = jnp.where(inr_vi, off_vi, FB)
        slot_vj = jnp.where(inr_vj, off_vj, FB)
        bias_i = plsc.load_gather(bslab_i, [slot_vi, iv & (LANE_T - 1)])
        bias_j = plsc.load_gather(bslab_j, [slot_vj, jv & (LANE_T - 1)])
        acc = (jnp.where(inr_vi, bias_i, bias_fix_i)
               + jnp.where(inr_vj, bias_j, bias_fix_j))
        rows = lane_idx * L
        for d in range(L):
            acc = acc + plsc.load_gather(tmp_v, [rows + d])
        pred_v[pl.ds(g * L, L)] = acc

    pltpu.sync_copy(pred_v, pred_hbm.at[pl.ds(base, B_PER_W)])


_ROW_BLKS = 8
_ROWS = B // _ROW_BLKS


def _tc_outer_body(pred_ref, xij_ref, out_ref):
    xf = xij_ref[:, :].astype(jnp.float32)
    logx = jnp.log(xf)
    fx = jnp.where(xf >= X_MAX, jnp.float32(1.0),
                   jnp.exp(ALPHA * jnp.log(xf / X_MAX)))
    diff = pred_ref[:, :] - logx
    out_ref[:, 0, :] = fx * diff * diff


_tc_outer = pl.pallas_call(
    _tc_outer_body,
    grid=(_ROW_BLKS,),
    in_specs=[
        pl.BlockSpec((_ROWS, 1), lambda i: (i, 0)),
        pl.BlockSpec((1, B), lambda i: (0, 0)),
    ],
    out_specs=pl.BlockSpec((_ROWS, 1, B), lambda i: (i, 0, 0)),
    out_shape=jax.ShapeDtypeStruct((B, 1, B), jnp.float32),
)


def kernel(x, emb_i, emb_j, bi, bj):
    xij = x[:, 2]
    pred = _sc_pred(x.T, emb_i.T, emb_j.T, bi.T, bj.T)
    return _tc_outer(pred.reshape(B, 1), xij.reshape(1, B))
